# baseline (speedup 1.0000x reference)
import functools

import jax
import jax.numpy as jnp
from jax import lax
from jax.experimental import pallas as pl
from jax.experimental.pallas import tpu as pltpu
from jax.experimental.pallas import tpu_sc as plsc

N = 100000
F = 128
G = 512
L = 16
NC, NS = 2, 16
NW = NC * NS
C = 80
NCH = N // C
SB = 80
NSB = C // SB
GROUPS = C // L
KREG = F // L
ACC_ROWS = G // NS


def _sc_body(feats_hbm, seg_hbm, w_hbm, b_hbm, zer_hbm,
             aw_hbm, part_hbm,
             xbuf, wbuf, id0, id1, id2, id3, id4,
             awbuf, ptmp, wtmp, wvbuf, bvbuf, acc):
    cid = lax.axis_index("c")
    sid = lax.axis_index("s")
    wid = sid * NC + cid

    pltpu.sync_copy(w_hbm, wvbuf)
    pltpu.sync_copy(b_hbm, bvbuf)
    pltpu.sync_copy(zer_hbm.at[pl.ds(sid * ACC_ROWS, ACC_ROWS)],
                    acc.at[pl.ds(sid * ACC_ROWS, ACC_ROWS)])
    plsc.subcore_barrier()

    wregs = [wvbuf[pl.ds(k * L, L)] for k in range(KREG)]
    bvec = bvbuf[...]
    rowiota = lax.iota(jnp.int32, L)
    rowiota16 = rowiota * L
    ids = [id0, id1, id2, id3, id4]

    nchunks = (NCH - 1 - wid) // NW + 1

    def chunk_body(t, carry):
        c = wid + t * NW
        r0 = c * C
        pltpu.sync_copy(feats_hbm.at[pl.ds(r0, C)], xbuf)
        for k in range(NSB):
            pltpu.sync_copy(seg_hbm.at[pl.ds(r0 + k * SB, SB)], ids[k])

        def group_body(g, carry2):
            base = g * L
            for rr in range(L):
                row = base + rr
                p = xbuf[row, pl.ds(0, L)] * wregs[0]
                for k in range(1, KREG):
                    p = p + xbuf[row, pl.ds(k * L, L)] * wregs[k]
                ptmp[pl.ds(rr * L, L)] = p
            awv = plsc.load_gather(ptmp, [rowiota16])
            for l in range(1, L):
                awv = awv + plsc.load_gather(ptmp, [rowiota16 + l])
            aw_full = awv + bvec
            awbuf[pl.ds(base, L)] = aw_full
            gate = 1.0 / (1.0 + jnp.exp(-aw_full))
            wtmp[pl.ds(0, L)] = gate
            wtmp[pl.ds(L, L)] = gate
            for rr in range(L):
                row = base + rr
                wbc = plsc.load_gather(wtmp, [jnp.full((L,), L + rr, jnp.int32)])
                for k in range(KREG):
                    wbuf[row, pl.ds(k * L, L)] = (
                        xbuf[row, pl.ds(k * L, L)] * wbc)
            return carry2

        lax.fori_loop(0, GROUPS, group_body, 0)
        pltpu.sync_copy(awbuf, aw_hbm.at[pl.ds(r0, C)])
        for k in range(NSB):
            pltpu.sync_copy(wbuf.at[pl.ds(k * SB, SB)], acc.at[ids[k]],
                            add=True)
        return carry

    lax.fori_loop(0, nchunks, chunk_body, 0)

    plsc.subcore_barrier()
    pltpu.sync_copy(
        acc.at[pl.ds(sid * ACC_ROWS, ACC_ROWS)],
        part_hbm.at[pl.ds(cid * G + sid * ACC_ROWS, ACC_ROWS)])


_sc_call = functools.partial(
    pl.kernel,
    _sc_body,
    out_type=[
        jax.ShapeDtypeStruct((N,), jnp.float32),
        jax.ShapeDtypeStruct((NC * G, F), jnp.float32),
    ],
    mesh=plsc.VectorSubcoreMesh(core_axis_name="c", subcore_axis_name="s"),
    compiler_params=pltpu.CompilerParams(needs_layout_passes=False),
    scratch_types=[
        pltpu.VMEM((C, F), jnp.float32),
        pltpu.VMEM((C, F), jnp.float32),
        pltpu.VMEM((SB,), jnp.int32),
        pltpu.VMEM((SB,), jnp.int32),
        pltpu.VMEM((SB,), jnp.int32),
        pltpu.VMEM((SB,), jnp.int32),
        pltpu.VMEM((SB,), jnp.int32),
        pltpu.VMEM((C,), jnp.float32),
        pltpu.VMEM((L * L,), jnp.float32),
        pltpu.VMEM((2 * L,), jnp.float32),
        pltpu.VMEM((F,), jnp.float32),
        pltpu.VMEM((L,), jnp.float32),
        pltpu.VMEM_SHARED((G, F), jnp.float32),
    ],
)()


def _combine_body(p_ref, o_ref):
    o_ref[...] = p_ref[0:G, :] + p_ref[G:2 * G, :]


def kernel(feats, segment_ids, W, b):
    seg = segment_ids.astype(jnp.int32)
    wcol = W.reshape(F)
    b16 = jnp.broadcast_to(b, (L,))
    zer = jnp.zeros((G, F), jnp.float32)
    aw_flat, part = _sc_call(feats, seg, wcol, b16, zer)
    out = pl.pallas_call(
        _combine_body,
        out_shape=jax.ShapeDtypeStruct((G, F), jnp.float32),
    )(part)
    return (out, aw_flat.reshape(N, 1))


# speedup vs baseline: 1.6229x; 1.6229x over previous
import functools

import jax
import jax.numpy as jnp
from jax import lax
from jax.experimental import pallas as pl
from jax.experimental.pallas import tpu as pltpu
from jax.experimental.pallas import tpu_sc as plsc

N = 100000
F = 128
G = 512
L = 16
NC, NS = 2, 16
NW = NC * NS
C = 160
NCH = N // C
SB = 80
NSB = C // SB
GROUPS = C // L
KREG = F // L
ACC_ROWS = G // NS
CPB = 20
PAD_IDS = NW * CPB * NSB


def _sc_body(feats_hbm, seg2_hbm, w_hbm, b_hbm, zer_hbm,
             aw_hbm, part_hbm,
             xb0, xb1, wb0, wb1, ab0, ab1, idb,
             ptmp, wtmp, wvbuf, bvbuf, acc,
             sem_ids, sem_in0, sem_in1, sem_sc0, sem_sc1, sem_aw0, sem_aw1):
    cid = lax.axis_index("c")
    sid = lax.axis_index("s")
    wid = sid * NC + cid
    c0 = wid * CPB
    nc = jnp.minimum(CPB, NCH - c0)

    xbufs, wbufs, awbufs = [xb0, xb1], [wb0, wb1], [ab0, ab1]
    sem_in, sem_sc, sem_aw = [sem_in0, sem_in1], [sem_sc0, sem_sc1], \
        [sem_aw0, sem_aw1]

    pltpu.async_copy(seg2_hbm.at[pl.ds(c0 * NSB, CPB * NSB)], idb, sem_ids)
    pltpu.sync_copy(w_hbm, wvbuf)
    pltpu.sync_copy(b_hbm, bvbuf)
    pltpu.sync_copy(zer_hbm.at[pl.ds(sid * ACC_ROWS, ACC_ROWS)],
                    acc.at[pl.ds(sid * ACC_ROWS, ACC_ROWS)])
    pltpu.make_async_copy(seg2_hbm.at[pl.ds(0, CPB * NSB)], idb,
                          sem_ids).wait()
    plsc.subcore_barrier()

    wregs = [wvbuf[pl.ds(k * L, L)] for k in range(KREG)]
    bvec = bvbuf[...]
    rowiota16 = lax.iota(jnp.int32, L) * L

    def start_in(t, p):
        r0 = (c0 + t) * C
        pltpu.async_copy(feats_hbm.at[pl.ds(r0, C)], xbufs[p], sem_in[p])

    def wait_in(p):
        pltpu.make_async_copy(feats_hbm.at[pl.ds(0, C)], xbufs[p],
                              sem_in[p]).wait()

    def start_out(t, p):
        r0 = (c0 + t) * C
        pltpu.async_copy(awbufs[p], aw_hbm.at[pl.ds(r0, C)], sem_aw[p])
        for k in range(NSB):
            pltpu.async_copy(wbufs[p].at[pl.ds(k * SB, SB)],
                             acc.at[idb.at[t * NSB + k]], sem_sc[p], add=True)

    def wait_out(t, p):
        pltpu.make_async_copy(awbufs[p], aw_hbm.at[pl.ds(0, C)],
                              sem_aw[p]).wait()
        for k in range(NSB):
            pltpu.make_async_copy(wbufs[p].at[pl.ds(k * SB, SB)],
                                  acc.at[idb.at[t * NSB + k]],
                                  sem_sc[p]).wait()

    def compute(p):
        xbuf, wbuf, awbuf = xbufs[p], wbufs[p], awbufs[p]

        def group_body(g, carry2):
            base = g * L
            for rr in range(L):
                row = base + rr
                pv = xbuf[row, pl.ds(0, L)] * wregs[0]
                for k in range(1, KREG):
                    pv = pv + xbuf[row, pl.ds(k * L, L)] * wregs[k]
                ptmp[pl.ds(rr * L, L)] = pv
            awv = plsc.load_gather(ptmp, [rowiota16])
            for l in range(1, L):
                awv = awv + plsc.load_gather(ptmp, [rowiota16 + l])
            aw_full = awv + bvec
            awbuf[pl.ds(base, L)] = aw_full
            gate = 1.0 / (1.0 + jnp.exp(-aw_full))
            wtmp[pl.ds(0, L)] = gate
            wtmp[pl.ds(L, L)] = gate
            for rr in range(L):
                row = base + rr
                wbc = plsc.load_gather(
                    wtmp, [jnp.full((L,), L + rr, jnp.int32)])
                for k in range(KREG):
                    wbuf[row, pl.ds(k * L, L)] = (
                        xbuf[row, pl.ds(k * L, L)] * wbc)
            return carry2

        lax.fori_loop(0, GROUPS, group_body, 0)

    start_in(0, 0)
    start_in(1, 1)

    def pair_body(u, carry):
        for p in range(2):
            t = 2 * u + p

            @pl.when(t < nc)
            def _():
                wait_in(p)

                @pl.when(t >= 2)
                def _():
                    wait_out(t - 2, p)

                compute(p)
                start_out(t, p)

                @pl.when(t + 2 < nc)
                def _():
                    start_in(t + 2, p)
        return carry

    lax.fori_loop(0, (nc + 1) // 2, pair_body, 0)

    for p in range(2):
        tp = jnp.where((nc - 1) % 2 == p, nc - 1, nc - 2)
        wait_out(tp, p)

    plsc.subcore_barrier()
    pltpu.sync_copy(
        acc.at[pl.ds(sid * ACC_ROWS, ACC_ROWS)],
        part_hbm.at[pl.ds(cid * G + sid * ACC_ROWS, ACC_ROWS)])


_sc_call = pl.kernel(
    _sc_body,
    out_type=[
        jax.ShapeDtypeStruct((N,), jnp.float32),
        jax.ShapeDtypeStruct((NC * G, F), jnp.float32),
    ],
    mesh=plsc.VectorSubcoreMesh(core_axis_name="c", subcore_axis_name="s"),
    compiler_params=pltpu.CompilerParams(needs_layout_passes=False),
    scratch_types=[
        pltpu.VMEM((C, F), jnp.float32),
        pltpu.VMEM((C, F), jnp.float32),
        pltpu.VMEM((C, F), jnp.float32),
        pltpu.VMEM((C, F), jnp.float32),
        pltpu.VMEM((C,), jnp.float32),
        pltpu.VMEM((C,), jnp.float32),
        pltpu.VMEM((CPB * NSB, SB), jnp.int32),
        pltpu.VMEM((L * L,), jnp.float32),
        pltpu.VMEM((2 * L,), jnp.float32),
        pltpu.VMEM((F,), jnp.float32),
        pltpu.VMEM((L,), jnp.float32),
        pltpu.VMEM_SHARED((G, F), jnp.float32),
        pltpu.SemaphoreType.DMA,
        pltpu.SemaphoreType.DMA,
        pltpu.SemaphoreType.DMA,
        pltpu.SemaphoreType.DMA,
        pltpu.SemaphoreType.DMA,
        pltpu.SemaphoreType.DMA,
        pltpu.SemaphoreType.DMA,
    ],
)


def _combine_body(p_ref, o_ref):
    o_ref[...] = p_ref[0:G, :] + p_ref[G:2 * G, :]


def kernel(feats, segment_ids, W, b):
    seg = segment_ids.astype(jnp.int32)
    seg2 = jnp.pad(seg, (0, PAD_IDS * SB - N)).reshape(PAD_IDS, SB)
    wcol = W.reshape(F)
    b16 = jnp.broadcast_to(b, (L,))
    zer = jnp.zeros((G, F), jnp.float32)
    aw_flat, part = _sc_call(feats, seg2, wcol, b16, zer)
    out = pl.pallas_call(
        _combine_body,
        out_shape=jax.ShapeDtypeStruct((G, F), jnp.float32),
    )(part)
    return (out, aw_flat.reshape(N, 1))


# speedup vs baseline: 2.3181x; 1.4283x over previous
import functools

import jax
import jax.numpy as jnp
from jax import lax
from jax.experimental import pallas as pl
from jax.experimental.pallas import tpu as pltpu
from jax.experimental.pallas import tpu_sc as plsc

N = 100000
F = 128
G = 512
L = 16
NC, NS = 2, 16
NW = NC * NS
C = 160
NCH = N // C
SB = 80
NSB = C // SB
GROUPS = C // L
KREG = F // L
ACC_ROWS = G // NS
CPB = 20
PAD_IDS = NW * CPB * NSB


def _sc_body(feats_hbm, seg2_hbm, w_hbm, b_hbm, zer_hbm,
             aw_hbm, part_hbm,
             xb0, xb1, wb0, wb1, ab0, ab1, idb,
             wvbuf, bvbuf, acc,
             sem_ids, sem_in0, sem_in1, sem_sc0, sem_sc1, sem_aw0, sem_aw1):
    cid = lax.axis_index("c")
    sid = lax.axis_index("s")
    wid = sid * NC + cid
    c0 = wid * CPB
    nc = jnp.minimum(CPB, NCH - c0)

    xbufs, wbufs, awbufs = [xb0, xb1], [wb0, wb1], [ab0, ab1]
    sem_in, sem_sc, sem_aw = [sem_in0, sem_in1], [sem_sc0, sem_sc1], \
        [sem_aw0, sem_aw1]

    pltpu.async_copy(seg2_hbm.at[pl.ds(c0 * NSB, CPB * NSB)], idb, sem_ids)
    pltpu.sync_copy(w_hbm, wvbuf)
    pltpu.sync_copy(b_hbm, bvbuf)
    pltpu.sync_copy(zer_hbm.at[pl.ds(sid * ACC_ROWS, ACC_ROWS)],
                    acc.at[pl.ds(sid * ACC_ROWS, ACC_ROWS)])
    pltpu.make_async_copy(seg2_hbm.at[pl.ds(0, CPB * NSB)], idb,
                          sem_ids).wait()
    plsc.subcore_barrier()

    wregs = [wvbuf[pl.ds(k * L, L)] for k in range(KREG)]
    bvec = bvbuf[...]
    lane_iota = lax.iota(jnp.int32, L)

    def start_in(t, p):
        r0 = (c0 + t) * C
        pltpu.async_copy(feats_hbm.at[pl.ds(r0, C)], xbufs[p], sem_in[p])

    def wait_in(p):
        pltpu.make_async_copy(feats_hbm.at[pl.ds(0, C)], xbufs[p],
                              sem_in[p]).wait()

    def start_out(t, p):
        r0 = (c0 + t) * C
        pltpu.async_copy(awbufs[p], aw_hbm.at[pl.ds(r0, C)], sem_aw[p])
        for k in range(NSB):
            pltpu.async_copy(wbufs[p].at[pl.ds(k * SB, SB)],
                             acc.at[idb.at[t * NSB + k]], sem_sc[p], add=True)

    def wait_out(t, p):
        pltpu.make_async_copy(awbufs[p], aw_hbm.at[pl.ds(0, C)],
                              sem_aw[p]).wait()
        for k in range(NSB):
            pltpu.make_async_copy(wbufs[p].at[pl.ds(k * SB, SB)],
                                  acc.at[idb.at[t * NSB + k]],
                                  sem_sc[p]).wait()

    def compute(p):
        xbuf, wbuf, awbuf = xbufs[p], wbufs[p], awbufs[p]

        def group_body(g, carry2):
            base = g * L
            awacc = bvec * 0.0
            for rr in range(L):
                row = base + rr
                xk = [xbuf[row, pl.ds(k * L, L)] for k in range(KREG)]
                pa = xk[0] * wregs[0]
                pb = xk[1] * wregs[1]
                for k in range(2, KREG, 2):
                    pa = pa + xk[k] * wregs[k]
                    pb = pb + xk[k + 1] * wregs[k + 1]
                s = jnp.sum(pa + pb)
                sv = lax.broadcast(s, (L,))
                awv = sv + bvec
                awacc = jnp.where(lane_iota == rr, awv, awacc)
                gate = 1.0 / (1.0 + jnp.exp(-awv))
                for k in range(KREG):
                    wbuf[row, pl.ds(k * L, L)] = xk[k] * gate
            awbuf[pl.ds(base, L)] = awacc
            return carry2

        lax.fori_loop(0, GROUPS, group_body, 0)

    start_in(0, 0)
    start_in(1, 1)

    def pair_body(u, carry):
        for p in range(2):
            t = 2 * u + p

            @pl.when(t < nc)
            def _():
                wait_in(p)

                @pl.when(t >= 2)
                def _():
                    wait_out(t - 2, p)

                compute(p)
                start_out(t, p)

                @pl.when(t + 2 < nc)
                def _():
                    start_in(t + 2, p)
        return carry

    lax.fori_loop(0, (nc + 1) // 2, pair_body, 0)

    for p in range(2):
        tp = jnp.where((nc - 1) % 2 == p, nc - 1, nc - 2)
        wait_out(tp, p)

    plsc.subcore_barrier()
    pltpu.sync_copy(
        acc.at[pl.ds(sid * ACC_ROWS, ACC_ROWS)],
        part_hbm.at[pl.ds(cid * G + sid * ACC_ROWS, ACC_ROWS)])


_sc_call = pl.kernel(
    _sc_body,
    out_type=[
        jax.ShapeDtypeStruct((N,), jnp.float32),
        jax.ShapeDtypeStruct((NC * G, F), jnp.float32),
    ],
    mesh=plsc.VectorSubcoreMesh(core_axis_name="c", subcore_axis_name="s"),
    compiler_params=pltpu.CompilerParams(needs_layout_passes=False),
    scratch_types=[
        pltpu.VMEM((C, F), jnp.float32),
        pltpu.VMEM((C, F), jnp.float32),
        pltpu.VMEM((C, F), jnp.float32),
        pltpu.VMEM((C, F), jnp.float32),
        pltpu.VMEM((C,), jnp.float32),
        pltpu.VMEM((C,), jnp.float32),
        pltpu.VMEM((CPB * NSB, SB), jnp.int32),
        pltpu.VMEM((F,), jnp.float32),
        pltpu.VMEM((L,), jnp.float32),
        pltpu.VMEM_SHARED((G, F), jnp.float32),
        pltpu.SemaphoreType.DMA,
        pltpu.SemaphoreType.DMA,
        pltpu.SemaphoreType.DMA,
        pltpu.SemaphoreType.DMA,
        pltpu.SemaphoreType.DMA,
        pltpu.SemaphoreType.DMA,
        pltpu.SemaphoreType.DMA,
    ],
)


def _combine_body(p_ref, o_ref):
    o_ref[...] = p_ref[0:G, :] + p_ref[G:2 * G, :]


def kernel(feats, segment_ids, W, b):
    seg = segment_ids.astype(jnp.int32)
    seg2 = jnp.pad(seg, (0, PAD_IDS * SB - N)).reshape(PAD_IDS, SB)
    wcol = W.reshape(F)
    b16 = jnp.broadcast_to(b, (L,))
    zer = jnp.zeros((G, F), jnp.float32)
    aw_flat, part = _sc_call(feats, seg2, wcol, b16, zer)
    out = pl.pallas_call(
        _combine_body,
        out_shape=jax.ShapeDtypeStruct((G, F), jnp.float32),
    )(part)
    return (out, aw_flat.reshape(N, 1))
